# baseline (speedup 1.0000x reference)
import functools

import jax
import jax.numpy as jnp
from jax import lax
from jax.experimental import pallas as pl
from jax.experimental.pallas import tpu as pltpu
from jax.experimental.pallas import tpu_sc as plsc

CBLK = 8192
N_FIELDS = 60082
B_SZ = 16
TOT = B_SZ * N_FIELDS
CPT = 30048
NSLOT = 1024


def _dense_body(ste_ref, pw_ref, uo_ref, out_ref, s_ref):
    s = jnp.dot(ste_ref[...], pw_ref[...], preferred_element_type=jnp.float32)
    s_ref[...] = s
    out_ref[...] = uo_ref[...] + s[:, 0][None, :]


_sc_mesh = plsc.VectorSubcoreMesh(core_axis_name="c", subcore_axis_name="s")


@functools.partial(
    pl.kernel,
    out_type=jax.ShapeDtypeStruct((TOT,), jnp.float32),
    mesh=_sc_mesh,
    compiler_params=pltpu.CompilerParams(needs_layout_passes=False),
    scratch_types=[
        pltpu.VMEM((CPT,), jnp.float32),
        pltpu.VMEM((NSLOT,), jnp.int32),
        pltpu.VMEM((NSLOT,), jnp.float32),
    ],
)
def _sc_scatter(dense_hbm, fidx_hbm, val_hbm, out_hbm, buf, fidx_v, val_v):
    wid = lax.axis_index("s") * 2 + lax.axis_index("c")
    lo = jnp.minimum(wid * CPT, TOT - CPT)
    pltpu.sync_copy(dense_hbm.at[pl.ds(lo, CPT)], buf)
    pltpu.sync_copy(fidx_hbm, fidx_v)
    pltpu.sync_copy(val_hbm, val_v)
    for g in range(NSLOT // 16):
        fi = fidx_v[pl.ds(g * 16, 16)]
        va = val_v[pl.ds(g * 16, 16)]
        m = (fi >= lo) & (fi < lo + CPT)
        plsc.store_scatter(buf, [fi - lo], va, mask=m)
    pltpu.sync_copy(buf, out_hbm.at[pl.ds(lo, CPT)])


def kernel(user_embedding, station_embedding, nodes, user_id, raw_field_embed,
           user_emb_table, station_emb_table, proj_W, proj_b, theta, alpha_fields,
           fcs_W, fcs_b, fcu_W, fcu_b, mh_W1, mh_b1, mh_W2, mh_b2):
    N, D = station_emb_table.shape
    B, _, K = nodes.shape

    w = fcs_W[:, 0]
    pw = proj_W @ w
    pbw = proj_b @ w
    c0 = pbw + fcs_b[0]

    th = theta[user_id, 0]
    user_mem = (1.0 - th)[:, None] * user_embedding + th[:, None] * user_emb_table[user_id]
    u = user_mem @ fcu_W[:, 0] + fcu_b[0]
    uo = u + c0

    his = nodes[:, 0, :]
    now = nodes[:, 1, :]
    t_idx = jnp.concatenate([now, his], axis=1)
    a_t = alpha_fields[t_idx, 0]
    a_now = a_t[:, :K]
    a_his = a_t[:, K:]

    in_now = (his[:, :, None] == now[:, None, :]).any(-1)
    coef_now = 1.0 - a_now * a_now
    coef_his = jnp.where(in_now, 1.0 - a_his * a_his, 1.0 - a_his)

    w2 = mh_W2 @ w
    h = jnp.einsum("bkd,dh->bkh", raw_field_embed[his], mh_W1) + mh_b1
    h = jax.nn.leaky_relu(h, negative_slope=0.01)
    mlp_d = h @ w2 + mh_b2 @ w

    coef = jnp.concatenate([coef_now, coef_his], axis=1)
    add = jnp.concatenate([jnp.zeros_like(coef_now), a_his * mlp_d], axis=1)
    addc = coef * pbw + add + fcs_b[0] + u[:, None]

    nb = pl.cdiv(N, CBLK)
    dense, s2d = pl.pallas_call(
        _dense_body,
        grid=(nb,),
        in_specs=[
            pl.BlockSpec((CBLK, D), lambda i: (i, 0)),
            pl.BlockSpec((D, 1), lambda i: (0, 0)),
            pl.BlockSpec((B, 1), lambda i: (0, 0)),
        ],
        out_specs=[
            pl.BlockSpec((B, CBLK), lambda i: (0, i)),
            pl.BlockSpec((CBLK, 1), lambda i: (i, 0)),
        ],
        out_shape=[
            jax.ShapeDtypeStruct((B, N), jnp.float32),
            jax.ShapeDtypeStruct((N, 1), jnp.float32),
        ],
    )(station_emb_table, pw[:, None], uo[:, None])

    s_at = s2d[t_idx, 0]
    val = (coef * s_at + addc).reshape(-1)
    fidx = (jnp.arange(B, dtype=jnp.int32)[:, None] * N + t_idx).reshape(-1)

    out_flat = _sc_scatter(dense.reshape(-1), fidx, val)
    return out_flat.reshape(B, N)


# speedup vs baseline: 1.0060x; 1.0060x over previous
import functools

import jax
import jax.numpy as jnp
from jax import lax
from jax.experimental import pallas as pl
from jax.experimental.pallas import tpu as pltpu
from jax.experimental.pallas import tpu_sc as plsc

N_FIELDS = 60082
NPAD = 61440
CBLK = 4096
B_SZ = 16
CCHUNK = 3840
NSLOT = 1024


def _dense_body(ste_ref, pw_ref, uo_ref, out_ref, s_ref):
    s = jnp.dot(ste_ref[...], pw_ref[...], preferred_element_type=jnp.float32)
    s_ref[...] = s
    out_ref[...] = uo_ref[...] + s[:, 0][None, :]


_sc_mesh = plsc.VectorSubcoreMesh(core_axis_name="c", subcore_axis_name="s")


@functools.partial(
    pl.kernel,
    out_type=jax.ShapeDtypeStruct((B_SZ, NPAD), jnp.float32),
    mesh=_sc_mesh,
    compiler_params=pltpu.CompilerParams(needs_layout_passes=False),
    scratch_types=[
        pltpu.VMEM((8, CCHUNK), jnp.float32),
        pltpu.VMEM((NSLOT,), jnp.int32),
        pltpu.VMEM((NSLOT,), jnp.int32),
        pltpu.VMEM((NSLOT,), jnp.float32),
    ],
)
def _sc_scatter(dense_hbm, rows_hbm, cols_hbm, val_hbm, out_hbm,
                buf, rows_v, cols_v, val_v):
    wid = lax.axis_index("s") * 2 + lax.axis_index("c")
    c = wid % 16
    r0 = (wid // 16) * 8
    clo = c * CCHUNK
    pltpu.sync_copy(dense_hbm.at[pl.ds(r0, 8), pl.ds(clo, CCHUNK)], buf)
    pltpu.sync_copy(rows_hbm, rows_v)
    pltpu.sync_copy(cols_hbm, cols_v)
    pltpu.sync_copy(val_hbm, val_v)
    for g in range(NSLOT // 16):
        ro = rows_v[pl.ds(g * 16, 16)]
        co = cols_v[pl.ds(g * 16, 16)]
        va = val_v[pl.ds(g * 16, 16)]
        m = (ro >= r0) & (ro < r0 + 8) & (co >= clo) & (co < clo + CCHUNK)
        plsc.store_scatter(buf, [ro - r0, co - clo], va, mask=m)
    pltpu.sync_copy(buf, out_hbm.at[pl.ds(r0, 8), pl.ds(clo, CCHUNK)])


def kernel(user_embedding, station_embedding, nodes, user_id, raw_field_embed,
           user_emb_table, station_emb_table, proj_W, proj_b, theta, alpha_fields,
           fcs_W, fcs_b, fcu_W, fcu_b, mh_W1, mh_b1, mh_W2, mh_b2):
    N, D = station_emb_table.shape
    B, _, K = nodes.shape

    w = fcs_W[:, 0]
    pw = proj_W @ w
    pbw = proj_b @ w
    c0 = pbw + fcs_b[0]

    th = theta[user_id, 0]
    user_mem = (1.0 - th)[:, None] * user_embedding + th[:, None] * user_emb_table[user_id]
    u = user_mem @ fcu_W[:, 0] + fcu_b[0]
    uo = u + c0

    his = nodes[:, 0, :]
    now = nodes[:, 1, :]
    t_idx = jnp.concatenate([now, his], axis=1)
    a_t = alpha_fields[t_idx, 0]
    a_now = a_t[:, :K]
    a_his = a_t[:, K:]

    in_now = (his[:, :, None] == now[:, None, :]).any(-1)
    coef_now = 1.0 - a_now * a_now
    coef_his = jnp.where(in_now, 1.0 - a_his * a_his, 1.0 - a_his)

    w2 = mh_W2 @ w
    h = jnp.einsum("bkd,dh->bkh", raw_field_embed[his], mh_W1) + mh_b1
    h = jax.nn.leaky_relu(h, negative_slope=0.01)
    mlp_d = h @ w2 + mh_b2 @ w

    coef = jnp.concatenate([coef_now, coef_his], axis=1)
    add = jnp.concatenate([jnp.zeros_like(coef_now), a_his * mlp_d], axis=1)
    addc = coef * pbw + add + fcs_b[0] + u[:, None]

    dense, s2d = pl.pallas_call(
        _dense_body,
        grid=(NPAD // CBLK,),
        in_specs=[
            pl.BlockSpec((CBLK, D), lambda i: (i, 0)),
            pl.BlockSpec((D, 1), lambda i: (0, 0)),
            pl.BlockSpec((B, 1), lambda i: (0, 0)),
        ],
        out_specs=[
            pl.BlockSpec((B, CBLK), lambda i: (0, i)),
            pl.BlockSpec((CBLK, 1), lambda i: (i, 0)),
        ],
        out_shape=[
            jax.ShapeDtypeStruct((B, NPAD), jnp.float32),
            jax.ShapeDtypeStruct((NPAD, 1), jnp.float32),
        ],
    )(station_emb_table, pw[:, None], uo[:, None])

    s_at = s2d[t_idx, 0]
    val = (coef * s_at + addc).reshape(-1)
    rows = jnp.broadcast_to(jnp.arange(B, dtype=jnp.int32)[:, None], (B, 2 * K)).reshape(-1)
    cols = t_idx.reshape(-1)

    out_pad = _sc_scatter(dense, rows, cols, val)
    return out_pad[:, :N]
